# baseline (speedup 1.0000x reference)
import functools

import jax
import jax.numpy as jnp
from jax import lax
from jax.experimental import pallas as pl
from jax.experimental.pallas import tpu as pltpu
from jax.experimental.pallas import tpu_sc as plsc

_N = 10000
_E = 320000
_D = 128
_ED = 16
_EPS = 1e-5

_NC = 2
_NS = 16
_GW = 128
_SW = 128
_EB = 512


def _sc_gather(table, idx_flat):
    n_idx = idx_flat.shape[0]
    width = table.shape[1]
    idx2 = idx_flat.reshape(1, n_idx)
    mesh = plsc.VectorSubcoreMesh(core_axis_name="c", subcore_axis_name="s")

    @functools.partial(
        pl.kernel,
        out_type=jax.ShapeDtypeStruct((n_idx, width), table.dtype),
        mesh=mesh,
        compiler_params=pltpu.CompilerParams(use_tc_tiling_on_sc=False),
    )
    def k(tab_hbm, i_hbm, o_hbm):
        def body(i_vmem, o_vmem):
            pltpu.sync_copy(tab_hbm.at[i_vmem.at[0]], o_vmem)

        pltpu.emit_pipeline(
            body,
            grid=(n_idx // _GW,),
            in_specs=[pl.BlockSpec((1, _GW), lambda i: (0, i))],
            out_specs=[pl.BlockSpec((_GW, width), lambda i: (i, 0))],
            core_axis_name=("c", "s"),
            dimension_semantics=(pltpu.PARALLEL,),
        )(i_hbm, o_hbm)

    return k(table, idx2)


def _pack_rows(x):
    xb = x.astype(jnp.bfloat16).astype(jnp.float32)
    bl = lax.bitcast_convert_type(xb[:, : _D // 2], jnp.int32)
    bh = lax.bitcast_convert_type(xb[:, _D // 2:], jnp.int32)
    return lax.shift_right_logical(bl, 16) | bh


def _unpack_halves(w):
    lo = lax.bitcast_convert_type(w << 16, jnp.float32)
    hi = lax.bitcast_convert_type(w & jnp.int32(-65536), jnp.float32)
    return lo, hi


def _sc_scatter_add(msg, dst, zeros_nd):
    dst2 = dst.reshape(1, _E)
    mesh = plsc.VectorSubcoreMesh(core_axis_name="c", subcore_axis_name="s")
    stripe = 624

    @functools.partial(
        pl.kernel,
        out_type=jax.ShapeDtypeStruct((_NC, _N, _D), jnp.float32),
        mesh=mesh,
        scratch_types=[pltpu.VMEM_SHARED((_N, _D), jnp.float32)],
    )
    def k(msg_hbm, dst_hbm, z_hbm, o_hbm, acc):
        cid = lax.axis_index("c")
        sid = lax.axis_index("s")
        r0 = sid * stripe
        pltpu.sync_copy(z_hbm.at[pl.ds(r0, stripe)], acc.at[pl.ds(r0, stripe)])

        @pl.when(sid == _NS - 1)
        def _():
            t0 = _NS * stripe
            pltpu.sync_copy(
                z_hbm.at[pl.ds(t0, _N - _NS * stripe)],
                acc.at[pl.ds(t0, _N - _NS * stripe)],
            )

        plsc.subcore_barrier()

        def body(msg_vmem, dst_vmem):
            pltpu.sync_copy(msg_vmem, acc.at[dst_vmem.at[0]], add=True)

        pltpu.emit_pipeline(
            body,
            grid=(_E // _SW,),
            in_specs=[
                pl.BlockSpec((_SW, _D), lambda i: (i, 0)),
                pl.BlockSpec((1, _SW), lambda i: (0, i)),
            ],
            out_specs=[],
            core_axis_name=("c", "s"),
            dimension_semantics=(pltpu.PARALLEL,),
        )(msg_hbm, dst_hbm)
        plsc.subcore_barrier()
        pltpu.sync_copy(
            acc.at[pl.ds(r0, stripe)], o_hbm.at[cid].at[pl.ds(r0, stripe)]
        )

        @pl.when(sid == _NS - 1)
        def _():
            t0 = _NS * stripe
            pltpu.sync_copy(
                acc.at[pl.ds(t0, _N - _NS * stripe)],
                o_hbm.at[cid].at[pl.ds(t0, _N - _NS * stripe)],
            )

    return k(msg, dst2, zeros_nd)


def _build_table(nf, w1dT, b1, interpret=False):
    def body(nf_ref, w_ref, b_ref, out_ref):
        x = nf_ref[...]
        out_ref[pl.ds(0, _N), :] = _pack_rows(x)
        pd = (
            jnp.dot(x.astype(jnp.bfloat16), w_ref[...],
                    preferred_element_type=jnp.float32)
            + b_ref[...]
        )
        out_ref[pl.ds(_N, _N), :] = _pack_rows(pd)

    return pl.pallas_call(
        body,
        out_shape=jax.ShapeDtypeStruct((2 * _N, _D // 2), jnp.int32),
        interpret=interpret,
    )(nf, w1dT.astype(jnp.bfloat16), b1)


def _edge_mlp(gathered, ef, w1sT, w1eT, w2T, b2, interpret=False):
    nb = _E // _EB

    hd = _D // 2

    def body(src_ref, pd_ref, ef_ref, w1s_ref, w1e_ref, w2_ref, b2_ref,
             out_ref):
        s_lo, s_hi = _unpack_halves(src_ref[...])
        p_lo, p_hi = _unpack_halves(pd_ref[...])
        slb = s_lo.astype(jnp.bfloat16)
        shb = s_hi.astype(jnp.bfloat16)
        efb = ef_ref[...].astype(jnp.bfloat16)

        def l1(c0):
            return (
                jnp.dot(slb, w1s_ref[pl.ds(0, hd), pl.ds(c0, hd)],
                        preferred_element_type=jnp.float32)
                + jnp.dot(shb, w1s_ref[pl.ds(hd, hd), pl.ds(c0, hd)],
                          preferred_element_type=jnp.float32)
                + jnp.dot(efb, w1e_ref[:, pl.ds(c0, hd)],
                          preferred_element_type=jnp.float32)
            )

        ha = jnp.maximum(l1(0) + p_lo, 0.0).astype(jnp.bfloat16)
        hb = jnp.maximum(l1(hd) + p_hi, 0.0).astype(jnp.bfloat16)

        def l2(c0):
            return (
                jnp.dot(ha, w2_ref[pl.ds(0, hd), pl.ds(c0, hd)],
                        preferred_element_type=jnp.float32)
                + jnp.dot(hb, w2_ref[pl.ds(hd, hd), pl.ds(c0, hd)],
                          preferred_element_type=jnp.float32)
                + b2_ref[:, pl.ds(c0, hd)]
            )

        out_ref[:, pl.ds(0, hd)] = s_lo * jax.nn.sigmoid(l2(0))
        out_ref[:, pl.ds(hd, hd)] = s_hi * jax.nn.sigmoid(l2(hd))

    return pl.pallas_call(
        body,
        grid=(nb,),
        in_specs=[
            pl.BlockSpec((_EB, _D // 2), lambda i: (i, 0)),
            pl.BlockSpec((_EB, _D // 2), lambda i: (i + nb, 0)),
            pl.BlockSpec((_EB, _ED), lambda i: (i, 0)),
            pl.BlockSpec((_D, _D), lambda i: (0, 0)),
            pl.BlockSpec((_ED, _D), lambda i: (0, 0)),
            pl.BlockSpec((_D, _D), lambda i: (0, 0)),
            pl.BlockSpec((1, _D), lambda i: (0, 0)),
        ],
        out_specs=pl.BlockSpec((_EB, _D), lambda i: (i, 0)),
        out_shape=jax.ShapeDtypeStruct((_E, _D), jnp.float32),
        interpret=interpret,
    )(gathered, gathered, ef, w1sT, w1eT, w2T, b2)


def _node_mlp(nf, agg2, w1aT, w1bT, b1, w2T, b2, gamma, beta, interpret=False):
    def body(nf_ref, agg_ref, wa_ref, wb_ref, b1_ref, w2_ref, b2_ref, g_ref,
             be_ref, out_ref):
        x = nf_ref[...]
        agg = agg_ref[0] + agg_ref[1]
        u = jnp.maximum(
            jnp.dot(x, wa_ref[...], preferred_element_type=jnp.float32)
            + jnp.dot(agg, wb_ref[...], preferred_element_type=jnp.float32)
            + b1_ref[...],
            0.0,
        )
        u = jnp.dot(u, w2_ref[...], preferred_element_type=jnp.float32) + b2_ref[...]
        mean = jnp.mean(u, axis=0, keepdims=True)
        cen = u - mean
        var = jnp.mean(cen * cen, axis=0, keepdims=True)
        u = g_ref[...] * cen * lax.rsqrt(var + _EPS) + be_ref[...]
        out_ref[...] = x + u

    return pl.pallas_call(
        body,
        out_shape=jax.ShapeDtypeStruct((_N, _D), jnp.float32),
        interpret=interpret,
    )(nf, agg2, w1aT, w1bT, b1, w2T, b2, gamma, beta)


def kernel(node_features, edge_index, edge_features, eW1, eb1, eW2, eb2,
           nW1, nb1, nW2, nb2, gamma, beta):
    src = edge_index[0]
    dst = edge_index[1]
    idx_flat = jnp.concatenate([src, dst + _N])

    w1sT = eW1[:, :_D].T.astype(jnp.bfloat16)
    w1dT = eW1[:, _D:2 * _D].T
    w1eT = eW1[:, 2 * _D:].T.astype(jnp.bfloat16)
    table = _build_table(node_features, w1dT, eb1.reshape(1, _D))
    gathered = _sc_gather(table, idx_flat)

    msg = _edge_mlp(
        gathered, edge_features, w1sT, w1eT,
        eW2.T.astype(jnp.bfloat16), eb2.reshape(1, _D),
    )

    zeros_nd = jnp.zeros((_N, _D), jnp.float32)
    agg2 = _sc_scatter_add(msg, dst, zeros_nd)

    w1aT = nW1[:, :_D].T
    w1bT = nW1[:, _D:].T
    return _node_mlp(
        node_features, agg2, w1aT, w1bT,
        nb1.reshape(1, _D), nW2.T, nb2.reshape(1, _D),
        gamma.reshape(1, _D), beta.reshape(1, _D),
    )


# speedup vs baseline: 1.3338x; 1.3338x over previous
import functools

import jax
import jax.numpy as jnp
from jax import lax
from jax.experimental import pallas as pl
from jax.experimental.pallas import tpu as pltpu
from jax.experimental.pallas import tpu_sc as plsc

_N = 10000
_E = 320000
_D = 128
_ED = 16
_EPS = 1e-5

_NC = 2
_NS = 16
_GW = 128
_SW = 128
_EB = 2048


def _sc_gather(table, idx_flat):
    n_idx = idx_flat.shape[0]
    width = table.shape[1]
    idx2 = idx_flat.reshape(1, n_idx)
    mesh = plsc.VectorSubcoreMesh(core_axis_name="c", subcore_axis_name="s")

    @functools.partial(
        pl.kernel,
        out_type=jax.ShapeDtypeStruct((n_idx, width), table.dtype),
        mesh=mesh,
        compiler_params=pltpu.CompilerParams(use_tc_tiling_on_sc=False),
    )
    def k(tab_hbm, i_hbm, o_hbm):
        def body(i_vmem, o_vmem):
            pltpu.sync_copy(tab_hbm.at[i_vmem.at[0]], o_vmem)

        pltpu.emit_pipeline(
            body,
            grid=(n_idx // _GW,),
            in_specs=[pl.BlockSpec((1, _GW), lambda i: (0, i))],
            out_specs=[pl.BlockSpec((_GW, width), lambda i: (i, 0))],
            core_axis_name=("c", "s"),
            dimension_semantics=(pltpu.PARALLEL,),
        )(i_hbm, o_hbm)

    return k(table, idx2)


def _pack_rows(x):
    xb = x.astype(jnp.bfloat16).astype(jnp.float32)
    bl = lax.bitcast_convert_type(xb[:, : _D // 2], jnp.int32)
    bh = lax.bitcast_convert_type(xb[:, _D // 2:], jnp.int32)
    return lax.shift_right_logical(bl, 16) | bh


def _unpack_halves(w):
    lo = lax.bitcast_convert_type(w << 16, jnp.float32)
    hi = lax.bitcast_convert_type(w & jnp.int32(-65536), jnp.float32)
    return lo, hi


def _sc_scatter_add(msg, dst, zeros_nd):
    dst2 = dst.reshape(1, _E)
    mesh = plsc.VectorSubcoreMesh(core_axis_name="c", subcore_axis_name="s")
    stripe = 624

    @functools.partial(
        pl.kernel,
        out_type=jax.ShapeDtypeStruct((_NC, _N, _D), jnp.float32),
        mesh=mesh,
        scratch_types=[pltpu.VMEM_SHARED((_N, _D), jnp.float32)],
    )
    def k(msg_hbm, dst_hbm, z_hbm, o_hbm, acc):
        cid = lax.axis_index("c")
        sid = lax.axis_index("s")
        r0 = sid * stripe
        pltpu.sync_copy(z_hbm.at[pl.ds(r0, stripe)], acc.at[pl.ds(r0, stripe)])

        @pl.when(sid == _NS - 1)
        def _():
            t0 = _NS * stripe
            pltpu.sync_copy(
                z_hbm.at[pl.ds(t0, _N - _NS * stripe)],
                acc.at[pl.ds(t0, _N - _NS * stripe)],
            )

        plsc.subcore_barrier()

        def body(msg_vmem, dst_vmem):
            pltpu.sync_copy(msg_vmem, acc.at[dst_vmem.at[0]], add=True)

        pltpu.emit_pipeline(
            body,
            grid=(_E // _SW,),
            in_specs=[
                pl.BlockSpec((_SW, _D), lambda i: (i, 0)),
                pl.BlockSpec((1, _SW), lambda i: (0, i)),
            ],
            out_specs=[],
            core_axis_name=("c", "s"),
            dimension_semantics=(pltpu.PARALLEL,),
        )(msg_hbm, dst_hbm)
        plsc.subcore_barrier()
        pltpu.sync_copy(
            acc.at[pl.ds(r0, stripe)], o_hbm.at[cid].at[pl.ds(r0, stripe)]
        )

        @pl.when(sid == _NS - 1)
        def _():
            t0 = _NS * stripe
            pltpu.sync_copy(
                acc.at[pl.ds(t0, _N - _NS * stripe)],
                o_hbm.at[cid].at[pl.ds(t0, _N - _NS * stripe)],
            )

    return k(msg, dst2, zeros_nd)


def _build_table(nf, w1dT, b1, interpret=False):
    def body(nf_ref, w_ref, b_ref, out_ref):
        x = nf_ref[...]
        out_ref[pl.ds(0, _N), :] = _pack_rows(x)
        pd = (
            jnp.dot(x.astype(jnp.bfloat16), w_ref[...],
                    preferred_element_type=jnp.float32)
            + b_ref[...]
        )
        out_ref[pl.ds(_N, _N), :] = _pack_rows(pd)

    return pl.pallas_call(
        body,
        out_shape=jax.ShapeDtypeStruct((2 * _N, _D // 2), jnp.int32),
        interpret=interpret,
    )(nf, w1dT.astype(jnp.bfloat16), b1)


def _edge_mlp(gathered, ef, w1sT, w1eT, w2T, b2, interpret=False):
    nb = _E // _EB

    hd = _D // 2

    def body(src_ref, pd_ref, ef_ref, w1s_ref, w1e_ref, w2_ref, b2_ref,
             out_ref):
        s_lo, s_hi = _unpack_halves(src_ref[...])
        p_lo, p_hi = _unpack_halves(pd_ref[...])
        slb = s_lo.astype(jnp.bfloat16)
        shb = s_hi.astype(jnp.bfloat16)
        efb = ef_ref[...].astype(jnp.bfloat16)

        def l1(c0):
            return (
                jnp.dot(slb, w1s_ref[pl.ds(0, hd), pl.ds(c0, hd)],
                        preferred_element_type=jnp.float32)
                + jnp.dot(shb, w1s_ref[pl.ds(hd, hd), pl.ds(c0, hd)],
                          preferred_element_type=jnp.float32)
                + jnp.dot(efb, w1e_ref[:, pl.ds(c0, hd)],
                          preferred_element_type=jnp.float32)
            )

        ha = jnp.maximum(l1(0) + p_lo, 0.0).astype(jnp.bfloat16)
        hb = jnp.maximum(l1(hd) + p_hi, 0.0).astype(jnp.bfloat16)

        def l2(c0):
            return (
                jnp.dot(ha, w2_ref[pl.ds(0, hd), pl.ds(c0, hd)],
                        preferred_element_type=jnp.float32)
                + jnp.dot(hb, w2_ref[pl.ds(hd, hd), pl.ds(c0, hd)],
                          preferred_element_type=jnp.float32)
                + b2_ref[:, pl.ds(c0, hd)]
            )

        out_ref[:, pl.ds(0, hd)] = s_lo * jax.nn.sigmoid(l2(0))
        out_ref[:, pl.ds(hd, hd)] = s_hi * jax.nn.sigmoid(l2(hd))

    return pl.pallas_call(
        body,
        grid=(nb,),
        in_specs=[
            pl.BlockSpec((_EB, _D // 2), lambda i: (i, 0)),
            pl.BlockSpec((_EB, _D // 2), lambda i: (i + nb, 0)),
            pl.BlockSpec((_EB, _ED), lambda i: (i, 0)),
            pl.BlockSpec((_D, _D), lambda i: (0, 0)),
            pl.BlockSpec((_ED, _D), lambda i: (0, 0)),
            pl.BlockSpec((_D, _D), lambda i: (0, 0)),
            pl.BlockSpec((1, _D), lambda i: (0, 0)),
        ],
        out_specs=pl.BlockSpec((_EB, _D), lambda i: (i, 0)),
        out_shape=jax.ShapeDtypeStruct((_E, _D), jnp.float32),
        interpret=interpret,
    )(gathered, gathered, ef, w1sT, w1eT, w2T, b2)


def _node_mlp(nf, agg2, w1aT, w1bT, b1, w2T, b2, gamma, beta, interpret=False):
    def body(nf_ref, agg_ref, wa_ref, wb_ref, b1_ref, w2_ref, b2_ref, g_ref,
             be_ref, out_ref):
        x = nf_ref[...]
        agg = agg_ref[0] + agg_ref[1]
        u = jnp.maximum(
            jnp.dot(x, wa_ref[...], preferred_element_type=jnp.float32)
            + jnp.dot(agg, wb_ref[...], preferred_element_type=jnp.float32)
            + b1_ref[...],
            0.0,
        )
        u = jnp.dot(u, w2_ref[...], preferred_element_type=jnp.float32) + b2_ref[...]
        mean = jnp.mean(u, axis=0, keepdims=True)
        cen = u - mean
        var = jnp.mean(cen * cen, axis=0, keepdims=True)
        u = g_ref[...] * cen * lax.rsqrt(var + _EPS) + be_ref[...]
        out_ref[...] = x + u

    return pl.pallas_call(
        body,
        out_shape=jax.ShapeDtypeStruct((_N, _D), jnp.float32),
        interpret=interpret,
    )(nf, agg2, w1aT, w1bT, b1, w2T, b2, gamma, beta)


def kernel(node_features, edge_index, edge_features, eW1, eb1, eW2, eb2,
           nW1, nb1, nW2, nb2, gamma, beta):
    src = edge_index[0]
    dst = edge_index[1]
    idx_flat = jnp.concatenate([src, dst + _N])

    w1sT = eW1[:, :_D].T.astype(jnp.bfloat16)
    w1dT = eW1[:, _D:2 * _D].T
    w1eT = eW1[:, 2 * _D:].T.astype(jnp.bfloat16)
    table = _build_table(node_features, w1dT, eb1.reshape(1, _D))
    gathered = _sc_gather(table, idx_flat)

    msg = _edge_mlp(
        gathered, edge_features, w1sT, w1eT,
        eW2.T.astype(jnp.bfloat16), eb2.reshape(1, _D),
    )

    zeros_nd = jnp.zeros((_N, _D), jnp.float32)
    agg2 = _sc_scatter_add(msg, dst, zeros_nd)

    w1aT = nW1[:, :_D].T
    w1bT = nW1[:, _D:].T
    return _node_mlp(
        node_features, agg2, w1aT, w1bT,
        nb1.reshape(1, _D), nW2.T, nb2.reshape(1, _D),
        gamma.reshape(1, _D), beta.reshape(1, _D),
    )


# speedup vs baseline: 1.3820x; 1.0361x over previous
import functools

import jax
import jax.numpy as jnp
from jax import lax
from jax.experimental import pallas as pl
from jax.experimental.pallas import tpu as pltpu
from jax.experimental.pallas import tpu_sc as plsc

_N = 10000
_E = 320000
_D = 128
_ED = 16
_EPS = 1e-5

_NC = 2
_NS = 16
_GW = 128
_SW = 128
_EB = 3200


def _sc_gather(table, idx_flat):
    n_idx = idx_flat.shape[0]
    width = table.shape[1]
    idx2 = idx_flat.reshape(1, n_idx)
    mesh = plsc.VectorSubcoreMesh(core_axis_name="c", subcore_axis_name="s")

    @functools.partial(
        pl.kernel,
        out_type=jax.ShapeDtypeStruct((n_idx, width), table.dtype),
        mesh=mesh,
        compiler_params=pltpu.CompilerParams(use_tc_tiling_on_sc=False),
    )
    def k(tab_hbm, i_hbm, o_hbm):
        def body(i_vmem, o_vmem):
            pltpu.sync_copy(tab_hbm.at[i_vmem.at[0]], o_vmem)

        pltpu.emit_pipeline(
            body,
            grid=(n_idx // _GW,),
            in_specs=[pl.BlockSpec((1, _GW), lambda i: (0, i))],
            out_specs=[pl.BlockSpec((_GW, width), lambda i: (i, 0))],
            core_axis_name=("c", "s"),
            dimension_semantics=(pltpu.PARALLEL,),
        )(i_hbm, o_hbm)

    return k(table, idx2)


def _pack_rows(x):
    xb = x.astype(jnp.bfloat16).astype(jnp.float32)
    bl = lax.bitcast_convert_type(xb[:, : _D // 2], jnp.int32)
    bh = lax.bitcast_convert_type(xb[:, _D // 2:], jnp.int32)
    return lax.shift_right_logical(bl, 16) | bh


def _unpack_halves(w):
    lo = lax.bitcast_convert_type(w << 16, jnp.float32)
    hi = lax.bitcast_convert_type(w & jnp.int32(-65536), jnp.float32)
    return lo, hi


def _sc_scatter_add(msg, dst, zeros_nd):
    dst2 = dst.reshape(1, _E)
    mesh = plsc.VectorSubcoreMesh(core_axis_name="c", subcore_axis_name="s")
    stripe = 624

    @functools.partial(
        pl.kernel,
        out_type=jax.ShapeDtypeStruct((_NC, _N, _D), jnp.float32),
        mesh=mesh,
        scratch_types=[pltpu.VMEM_SHARED((_N, _D), jnp.float32)],
    )
    def k(msg_hbm, dst_hbm, z_hbm, o_hbm, acc):
        cid = lax.axis_index("c")
        sid = lax.axis_index("s")
        r0 = sid * stripe
        pltpu.sync_copy(z_hbm.at[pl.ds(r0, stripe)], acc.at[pl.ds(r0, stripe)])

        @pl.when(sid == _NS - 1)
        def _():
            t0 = _NS * stripe
            pltpu.sync_copy(
                z_hbm.at[pl.ds(t0, _N - _NS * stripe)],
                acc.at[pl.ds(t0, _N - _NS * stripe)],
            )

        plsc.subcore_barrier()

        def body(msg_vmem, dst_vmem):
            pltpu.sync_copy(msg_vmem, acc.at[dst_vmem.at[0]], add=True)

        pltpu.emit_pipeline(
            body,
            grid=(_E // _SW,),
            in_specs=[
                pl.BlockSpec((_SW, _D), lambda i: (i, 0)),
                pl.BlockSpec((1, _SW), lambda i: (0, i)),
            ],
            out_specs=[],
            core_axis_name=("c", "s"),
            dimension_semantics=(pltpu.PARALLEL,),
        )(msg_hbm, dst_hbm)
        plsc.subcore_barrier()
        pltpu.sync_copy(
            acc.at[pl.ds(r0, stripe)], o_hbm.at[cid].at[pl.ds(r0, stripe)]
        )

        @pl.when(sid == _NS - 1)
        def _():
            t0 = _NS * stripe
            pltpu.sync_copy(
                acc.at[pl.ds(t0, _N - _NS * stripe)],
                o_hbm.at[cid].at[pl.ds(t0, _N - _NS * stripe)],
            )

    return k(msg, dst2, zeros_nd)


def _build_table(nf, w1dT, b1, interpret=False):
    def body(nf_ref, w_ref, b_ref, out_ref):
        x = nf_ref[...]
        out_ref[pl.ds(0, _N), :] = _pack_rows(x)
        pd = (
            jnp.dot(x.astype(jnp.bfloat16), w_ref[...],
                    preferred_element_type=jnp.float32)
            + b_ref[...]
        )
        out_ref[pl.ds(_N, _N), :] = _pack_rows(pd)

    return pl.pallas_call(
        body,
        out_shape=jax.ShapeDtypeStruct((2 * _N, _D // 2), jnp.int32),
        interpret=interpret,
    )(nf, w1dT.astype(jnp.bfloat16), b1)


def _edge_mlp(gathered, ef, w1sT, w1eT, w2T, b2, interpret=False):
    nb = _E // _EB

    hd = _D // 2

    def body(src_ref, pd_ref, ef_ref, w1s_ref, w1e_ref, w2_ref, b2_ref,
             out_ref):
        s_lo, s_hi = _unpack_halves(src_ref[...])
        p_lo, p_hi = _unpack_halves(pd_ref[...])
        slb = s_lo.astype(jnp.bfloat16)
        shb = s_hi.astype(jnp.bfloat16)
        efb = ef_ref[...].astype(jnp.bfloat16)

        def l1(c0):
            return (
                jnp.dot(slb, w1s_ref[pl.ds(0, hd), pl.ds(c0, hd)],
                        preferred_element_type=jnp.float32)
                + jnp.dot(shb, w1s_ref[pl.ds(hd, hd), pl.ds(c0, hd)],
                          preferred_element_type=jnp.float32)
                + jnp.dot(efb, w1e_ref[:, pl.ds(c0, hd)],
                          preferred_element_type=jnp.float32)
            )

        ha = jnp.maximum(l1(0) + p_lo, 0.0).astype(jnp.bfloat16)
        hb = jnp.maximum(l1(hd) + p_hi, 0.0).astype(jnp.bfloat16)

        def l2(c0):
            return (
                jnp.dot(ha, w2_ref[pl.ds(0, hd), pl.ds(c0, hd)],
                        preferred_element_type=jnp.float32)
                + jnp.dot(hb, w2_ref[pl.ds(hd, hd), pl.ds(c0, hd)],
                          preferred_element_type=jnp.float32)
                + b2_ref[:, pl.ds(c0, hd)]
            )

        out_ref[:, pl.ds(0, hd)] = s_lo * jax.nn.sigmoid(l2(0))
        out_ref[:, pl.ds(hd, hd)] = s_hi * jax.nn.sigmoid(l2(hd))

    return pl.pallas_call(
        body,
        grid=(nb,),
        in_specs=[
            pl.BlockSpec((_EB, _D // 2), lambda i: (i, 0)),
            pl.BlockSpec((_EB, _D // 2), lambda i: (i + nb, 0)),
            pl.BlockSpec((_EB, _ED), lambda i: (i, 0)),
            pl.BlockSpec((_D, _D), lambda i: (0, 0)),
            pl.BlockSpec((_ED, _D), lambda i: (0, 0)),
            pl.BlockSpec((_D, _D), lambda i: (0, 0)),
            pl.BlockSpec((1, _D), lambda i: (0, 0)),
        ],
        out_specs=pl.BlockSpec((_EB, _D), lambda i: (i, 0)),
        out_shape=jax.ShapeDtypeStruct((_E, _D), jnp.float32),
        interpret=interpret,
    )(gathered, gathered, ef, w1sT, w1eT, w2T, b2)


def _node_mlp(nf, agg2, w1aT, w1bT, b1, w2T, b2, gamma, beta, interpret=False):
    def body(nf_ref, agg_ref, wa_ref, wb_ref, b1_ref, w2_ref, b2_ref, g_ref,
             be_ref, out_ref):
        x = nf_ref[...]
        agg = agg_ref[0] + agg_ref[1]
        u = jnp.maximum(
            jnp.dot(x, wa_ref[...], preferred_element_type=jnp.float32)
            + jnp.dot(agg, wb_ref[...], preferred_element_type=jnp.float32)
            + b1_ref[...],
            0.0,
        )
        u = jnp.dot(u, w2_ref[...], preferred_element_type=jnp.float32) + b2_ref[...]
        mean = jnp.mean(u, axis=0, keepdims=True)
        cen = u - mean
        var = jnp.mean(cen * cen, axis=0, keepdims=True)
        u = g_ref[...] * cen * lax.rsqrt(var + _EPS) + be_ref[...]
        out_ref[...] = x + u

    return pl.pallas_call(
        body,
        out_shape=jax.ShapeDtypeStruct((_N, _D), jnp.float32),
        interpret=interpret,
    )(nf, agg2, w1aT, w1bT, b1, w2T, b2, gamma, beta)


def kernel(node_features, edge_index, edge_features, eW1, eb1, eW2, eb2,
           nW1, nb1, nW2, nb2, gamma, beta):
    src = edge_index[0]
    dst = edge_index[1]
    idx_flat = jnp.concatenate([src, dst + _N])

    w1sT = eW1[:, :_D].T.astype(jnp.bfloat16)
    w1dT = eW1[:, _D:2 * _D].T
    w1eT = eW1[:, 2 * _D:].T.astype(jnp.bfloat16)
    table = _build_table(node_features, w1dT, eb1.reshape(1, _D))
    gathered = _sc_gather(table, idx_flat)

    msg = _edge_mlp(
        gathered, edge_features, w1sT, w1eT,
        eW2.T.astype(jnp.bfloat16), eb2.reshape(1, _D),
    )

    zeros_nd = jnp.zeros((_N, _D), jnp.float32)
    agg2 = _sc_scatter_add(msg, dst, zeros_nd)

    w1aT = nW1[:, :_D].T
    w1bT = nW1[:, _D:].T
    return _node_mlp(
        node_features, agg2, w1aT, w1bT,
        nb1.reshape(1, _D), nW2.T, nb2.reshape(1, _D),
        gamma.reshape(1, _D), beta.reshape(1, _D),
    )


# speedup vs baseline: 1.4121x; 1.0218x over previous
import functools

import jax
import jax.numpy as jnp
from jax import lax
from jax.experimental import pallas as pl
from jax.experimental.pallas import tpu as pltpu
from jax.experimental.pallas import tpu_sc as plsc

_N = 10000
_E = 320000
_D = 128
_ED = 16
_EPS = 1e-5

_NC = 2
_NS = 16
_GW = 128
_SW = 128
_EB = 6400


def _sc_gather(table, idx_flat):
    n_idx = idx_flat.shape[0]
    width = table.shape[1]
    idx2 = idx_flat.reshape(1, n_idx)
    mesh = plsc.VectorSubcoreMesh(core_axis_name="c", subcore_axis_name="s")

    @functools.partial(
        pl.kernel,
        out_type=jax.ShapeDtypeStruct((n_idx, width), table.dtype),
        mesh=mesh,
        compiler_params=pltpu.CompilerParams(use_tc_tiling_on_sc=False),
    )
    def k(tab_hbm, i_hbm, o_hbm):
        def body(i_vmem, o_vmem):
            pltpu.sync_copy(tab_hbm.at[i_vmem.at[0]], o_vmem)

        pltpu.emit_pipeline(
            body,
            grid=(n_idx // _GW,),
            in_specs=[pl.BlockSpec((1, _GW), lambda i: (0, i))],
            out_specs=[pl.BlockSpec((_GW, width), lambda i: (i, 0))],
            core_axis_name=("c", "s"),
            dimension_semantics=(pltpu.PARALLEL,),
        )(i_hbm, o_hbm)

    return k(table, idx2)


def _pack_rows(x):
    xb = x.astype(jnp.bfloat16).astype(jnp.float32)
    bl = lax.bitcast_convert_type(xb[:, : _D // 2], jnp.int32)
    bh = lax.bitcast_convert_type(xb[:, _D // 2:], jnp.int32)
    return lax.shift_right_logical(bl, 16) | bh


def _unpack_halves(w):
    lo = lax.bitcast_convert_type(w << 16, jnp.float32)
    hi = lax.bitcast_convert_type(w & jnp.int32(-65536), jnp.float32)
    return lo, hi


def _sc_scatter_add(msg, dst, zeros_nd):
    dst2 = dst.reshape(1, _E)
    mesh = plsc.VectorSubcoreMesh(core_axis_name="c", subcore_axis_name="s")
    stripe = 624

    @functools.partial(
        pl.kernel,
        out_type=jax.ShapeDtypeStruct((_NC, _N, _D), jnp.float32),
        mesh=mesh,
        scratch_types=[pltpu.VMEM_SHARED((_N, _D), jnp.float32)],
    )
    def k(msg_hbm, dst_hbm, z_hbm, o_hbm, acc):
        cid = lax.axis_index("c")
        sid = lax.axis_index("s")
        r0 = sid * stripe
        pltpu.sync_copy(z_hbm.at[pl.ds(r0, stripe)], acc.at[pl.ds(r0, stripe)])

        @pl.when(sid == _NS - 1)
        def _():
            t0 = _NS * stripe
            pltpu.sync_copy(
                z_hbm.at[pl.ds(t0, _N - _NS * stripe)],
                acc.at[pl.ds(t0, _N - _NS * stripe)],
            )

        plsc.subcore_barrier()

        def body(msg_vmem, dst_vmem):
            pltpu.sync_copy(msg_vmem, acc.at[dst_vmem.at[0]], add=True)

        pltpu.emit_pipeline(
            body,
            grid=(_E // _SW,),
            in_specs=[
                pl.BlockSpec((_SW, _D), lambda i: (i, 0)),
                pl.BlockSpec((1, _SW), lambda i: (0, i)),
            ],
            out_specs=[],
            core_axis_name=("c", "s"),
            dimension_semantics=(pltpu.PARALLEL,),
        )(msg_hbm, dst_hbm)
        plsc.subcore_barrier()
        pltpu.sync_copy(
            acc.at[pl.ds(r0, stripe)], o_hbm.at[cid].at[pl.ds(r0, stripe)]
        )

        @pl.when(sid == _NS - 1)
        def _():
            t0 = _NS * stripe
            pltpu.sync_copy(
                acc.at[pl.ds(t0, _N - _NS * stripe)],
                o_hbm.at[cid].at[pl.ds(t0, _N - _NS * stripe)],
            )

    return k(msg, dst2, zeros_nd)


def _build_table(nf, w1dT, b1, interpret=False):
    def body(nf_ref, w_ref, b_ref, out_ref):
        x = nf_ref[...]
        out_ref[pl.ds(0, _N), :] = _pack_rows(x)
        pd = (
            jnp.dot(x.astype(jnp.bfloat16), w_ref[...],
                    preferred_element_type=jnp.float32)
            + b_ref[...]
        )
        out_ref[pl.ds(_N, _N), :] = _pack_rows(pd)

    return pl.pallas_call(
        body,
        out_shape=jax.ShapeDtypeStruct((2 * _N, _D // 2), jnp.int32),
        interpret=interpret,
    )(nf, w1dT.astype(jnp.bfloat16), b1)


def _edge_mlp(gathered, ef, w1sT, w1eT, w2T, b2, interpret=False):
    nb = _E // _EB

    hd = _D // 2

    def body(src_ref, pd_ref, ef_ref, w1s_ref, w1e_ref, w2_ref, b2_ref,
             out_ref):
        s_lo, s_hi = _unpack_halves(src_ref[...])
        p_lo, p_hi = _unpack_halves(pd_ref[...])
        slb = s_lo.astype(jnp.bfloat16)
        shb = s_hi.astype(jnp.bfloat16)
        efb = ef_ref[...].astype(jnp.bfloat16)

        def l1(c0):
            return (
                jnp.dot(slb, w1s_ref[pl.ds(0, hd), pl.ds(c0, hd)],
                        preferred_element_type=jnp.float32)
                + jnp.dot(shb, w1s_ref[pl.ds(hd, hd), pl.ds(c0, hd)],
                          preferred_element_type=jnp.float32)
                + jnp.dot(efb, w1e_ref[:, pl.ds(c0, hd)],
                          preferred_element_type=jnp.float32)
            )

        ha = jnp.maximum(l1(0) + p_lo, 0.0).astype(jnp.bfloat16)
        hb = jnp.maximum(l1(hd) + p_hi, 0.0).astype(jnp.bfloat16)

        def l2(c0):
            return (
                jnp.dot(ha, w2_ref[pl.ds(0, hd), pl.ds(c0, hd)],
                        preferred_element_type=jnp.float32)
                + jnp.dot(hb, w2_ref[pl.ds(hd, hd), pl.ds(c0, hd)],
                          preferred_element_type=jnp.float32)
                + b2_ref[:, pl.ds(c0, hd)]
            )

        out_ref[:, pl.ds(0, hd)] = s_lo * jax.nn.sigmoid(l2(0))
        out_ref[:, pl.ds(hd, hd)] = s_hi * jax.nn.sigmoid(l2(hd))

    return pl.pallas_call(
        body,
        grid=(nb,),
        in_specs=[
            pl.BlockSpec((_EB, _D // 2), lambda i: (i, 0)),
            pl.BlockSpec((_EB, _D // 2), lambda i: (i + nb, 0)),
            pl.BlockSpec((_EB, _ED), lambda i: (i, 0)),
            pl.BlockSpec((_D, _D), lambda i: (0, 0)),
            pl.BlockSpec((_ED, _D), lambda i: (0, 0)),
            pl.BlockSpec((_D, _D), lambda i: (0, 0)),
            pl.BlockSpec((1, _D), lambda i: (0, 0)),
        ],
        out_specs=pl.BlockSpec((_EB, _D), lambda i: (i, 0)),
        out_shape=jax.ShapeDtypeStruct((_E, _D), jnp.float32),
        interpret=interpret,
    )(gathered, gathered, ef, w1sT, w1eT, w2T, b2)


def _node_mlp(nf, agg2, w1aT, w1bT, b1, w2T, b2, gamma, beta, interpret=False):
    def body(nf_ref, agg_ref, wa_ref, wb_ref, b1_ref, w2_ref, b2_ref, g_ref,
             be_ref, out_ref):
        x = nf_ref[...]
        agg = agg_ref[0] + agg_ref[1]
        u = jnp.maximum(
            jnp.dot(x, wa_ref[...], preferred_element_type=jnp.float32)
            + jnp.dot(agg, wb_ref[...], preferred_element_type=jnp.float32)
            + b1_ref[...],
            0.0,
        )
        u = jnp.dot(u, w2_ref[...], preferred_element_type=jnp.float32) + b2_ref[...]
        mean = jnp.mean(u, axis=0, keepdims=True)
        cen = u - mean
        var = jnp.mean(cen * cen, axis=0, keepdims=True)
        u = g_ref[...] * cen * lax.rsqrt(var + _EPS) + be_ref[...]
        out_ref[...] = x + u

    return pl.pallas_call(
        body,
        out_shape=jax.ShapeDtypeStruct((_N, _D), jnp.float32),
        interpret=interpret,
    )(nf, agg2, w1aT, w1bT, b1, w2T, b2, gamma, beta)


def kernel(node_features, edge_index, edge_features, eW1, eb1, eW2, eb2,
           nW1, nb1, nW2, nb2, gamma, beta):
    src = edge_index[0]
    dst = edge_index[1]
    idx_flat = jnp.concatenate([src, dst + _N])

    w1sT = eW1[:, :_D].T.astype(jnp.bfloat16)
    w1dT = eW1[:, _D:2 * _D].T
    w1eT = eW1[:, 2 * _D:].T.astype(jnp.bfloat16)
    table = _build_table(node_features, w1dT, eb1.reshape(1, _D))
    gathered = _sc_gather(table, idx_flat)

    msg = _edge_mlp(
        gathered, edge_features, w1sT, w1eT,
        eW2.T.astype(jnp.bfloat16), eb2.reshape(1, _D),
    )

    zeros_nd = jnp.zeros((_N, _D), jnp.float32)
    agg2 = _sc_scatter_add(msg, dst, zeros_nd)

    w1aT = nW1[:, :_D].T
    w1bT = nW1[:, _D:].T
    return _node_mlp(
        node_features, agg2, w1aT, w1bT,
        nb1.reshape(1, _D), nW2.T, nb2.reshape(1, _D),
        gamma.reshape(1, _D), beta.reshape(1, _D),
    )


# speedup vs baseline: 1.4753x; 1.0447x over previous
import functools

import jax
import jax.numpy as jnp
from jax import lax
from jax.experimental import pallas as pl
from jax.experimental.pallas import tpu as pltpu
from jax.experimental.pallas import tpu_sc as plsc

_N = 10000
_E = 320000
_D = 128
_ED = 16
_EPS = 1e-5

_NC = 2
_NS = 16
_GW = 256
_SW = 128
_EB = 8000


def _sc_gather(table, idx_flat):
    n_idx = idx_flat.shape[0]
    width = table.shape[1]
    idx2 = idx_flat.reshape(1, n_idx)
    mesh = plsc.VectorSubcoreMesh(core_axis_name="c", subcore_axis_name="s")

    @functools.partial(
        pl.kernel,
        out_type=jax.ShapeDtypeStruct((n_idx, width), table.dtype),
        mesh=mesh,
        compiler_params=pltpu.CompilerParams(use_tc_tiling_on_sc=False),
    )
    def k(tab_hbm, i_hbm, o_hbm):
        def body(i_vmem, o_vmem):
            pltpu.sync_copy(tab_hbm.at[i_vmem.at[0]], o_vmem)

        pltpu.emit_pipeline(
            body,
            grid=(n_idx // _GW,),
            in_specs=[pl.BlockSpec((1, _GW), lambda i: (0, i))],
            out_specs=[pl.BlockSpec((_GW, width), lambda i: (i, 0))],
            core_axis_name=("c", "s"),
            dimension_semantics=(pltpu.PARALLEL,),
        )(i_hbm, o_hbm)

    return k(table, idx2)


def _pack_rows(x):
    xb = x.astype(jnp.bfloat16).astype(jnp.float32)
    bl = lax.bitcast_convert_type(xb[:, : _D // 2], jnp.int32)
    bh = lax.bitcast_convert_type(xb[:, _D // 2:], jnp.int32)
    return lax.shift_right_logical(bl, 16) | bh


def _unpack_halves(w):
    lo = lax.bitcast_convert_type(w << 16, jnp.float32)
    hi = lax.bitcast_convert_type(w & jnp.int32(-65536), jnp.float32)
    return lo, hi


def _sc_scatter_add(msg, dst, zeros_nd):
    dst2 = dst.reshape(1, _E)
    mesh = plsc.VectorSubcoreMesh(core_axis_name="c", subcore_axis_name="s")
    stripe = 624

    @functools.partial(
        pl.kernel,
        out_type=jax.ShapeDtypeStruct((_NC, _N, _D), jnp.float32),
        mesh=mesh,
        scratch_types=[pltpu.VMEM_SHARED((_N, _D), jnp.float32)],
    )
    def k(msg_hbm, dst_hbm, z_hbm, o_hbm, acc):
        cid = lax.axis_index("c")
        sid = lax.axis_index("s")
        r0 = sid * stripe
        pltpu.sync_copy(z_hbm.at[pl.ds(r0, stripe)], acc.at[pl.ds(r0, stripe)])

        @pl.when(sid == _NS - 1)
        def _():
            t0 = _NS * stripe
            pltpu.sync_copy(
                z_hbm.at[pl.ds(t0, _N - _NS * stripe)],
                acc.at[pl.ds(t0, _N - _NS * stripe)],
            )

        plsc.subcore_barrier()

        def body(msg_vmem, dst_vmem):
            pltpu.sync_copy(msg_vmem, acc.at[dst_vmem.at[0]], add=True)

        pltpu.emit_pipeline(
            body,
            grid=(_E // _SW,),
            in_specs=[
                pl.BlockSpec((_SW, _D), lambda i: (i, 0)),
                pl.BlockSpec((1, _SW), lambda i: (0, i)),
            ],
            out_specs=[],
            core_axis_name=("c", "s"),
            dimension_semantics=(pltpu.PARALLEL,),
        )(msg_hbm, dst_hbm)
        plsc.subcore_barrier()
        pltpu.sync_copy(
            acc.at[pl.ds(r0, stripe)], o_hbm.at[cid].at[pl.ds(r0, stripe)]
        )

        @pl.when(sid == _NS - 1)
        def _():
            t0 = _NS * stripe
            pltpu.sync_copy(
                acc.at[pl.ds(t0, _N - _NS * stripe)],
                o_hbm.at[cid].at[pl.ds(t0, _N - _NS * stripe)],
            )

    return k(msg, dst2, zeros_nd)


def _build_table(nf, w1dT, b1, interpret=False):
    def body(nf_ref, w_ref, b_ref, out_ref):
        x = nf_ref[...]
        out_ref[pl.ds(0, _N), :] = _pack_rows(x)
        pd = (
            jnp.dot(x.astype(jnp.bfloat16), w_ref[...],
                    preferred_element_type=jnp.float32)
            + b_ref[...]
        )
        out_ref[pl.ds(_N, _N), :] = _pack_rows(pd)

    return pl.pallas_call(
        body,
        out_shape=jax.ShapeDtypeStruct((2 * _N, _D // 2), jnp.int32),
        interpret=interpret,
    )(nf, w1dT.astype(jnp.bfloat16), b1)


def _edge_mlp(gathered, ef, w1sT, w1eT, w2T, b2, interpret=False):
    nb = _E // _EB

    hd = _D // 2

    def body(src_ref, pd_ref, ef_ref, w1s_ref, w1e_ref, w2_ref, b2_ref,
             out_ref):
        s_lo, s_hi = _unpack_halves(src_ref[...])
        p_lo, p_hi = _unpack_halves(pd_ref[...])
        slb = s_lo.astype(jnp.bfloat16)
        shb = s_hi.astype(jnp.bfloat16)
        efb = ef_ref[...].astype(jnp.bfloat16)

        def l1(c0):
            return (
                jnp.dot(slb, w1s_ref[pl.ds(0, hd), pl.ds(c0, hd)],
                        preferred_element_type=jnp.float32)
                + jnp.dot(shb, w1s_ref[pl.ds(hd, hd), pl.ds(c0, hd)],
                          preferred_element_type=jnp.float32)
                + jnp.dot(efb, w1e_ref[:, pl.ds(c0, hd)],
                          preferred_element_type=jnp.float32)
            )

        ha = jnp.maximum(l1(0) + p_lo, 0.0).astype(jnp.bfloat16)
        hb = jnp.maximum(l1(hd) + p_hi, 0.0).astype(jnp.bfloat16)

        def l2(c0):
            return (
                jnp.dot(ha, w2_ref[pl.ds(0, hd), pl.ds(c0, hd)],
                        preferred_element_type=jnp.float32)
                + jnp.dot(hb, w2_ref[pl.ds(hd, hd), pl.ds(c0, hd)],
                          preferred_element_type=jnp.float32)
                + b2_ref[:, pl.ds(c0, hd)]
            )

        out_ref[:, pl.ds(0, hd)] = s_lo * jax.nn.sigmoid(l2(0))
        out_ref[:, pl.ds(hd, hd)] = s_hi * jax.nn.sigmoid(l2(hd))

    return pl.pallas_call(
        body,
        grid=(nb,),
        in_specs=[
            pl.BlockSpec((_EB, _D // 2), lambda i: (i, 0)),
            pl.BlockSpec((_EB, _D // 2), lambda i: (i + nb, 0)),
            pl.BlockSpec((_EB, _ED), lambda i: (i, 0)),
            pl.BlockSpec((_D, _D), lambda i: (0, 0)),
            pl.BlockSpec((_ED, _D), lambda i: (0, 0)),
            pl.BlockSpec((_D, _D), lambda i: (0, 0)),
            pl.BlockSpec((1, _D), lambda i: (0, 0)),
        ],
        out_specs=pl.BlockSpec((_EB, _D), lambda i: (i, 0)),
        out_shape=jax.ShapeDtypeStruct((_E, _D), jnp.float32),
        interpret=interpret,
    )(gathered, gathered, ef, w1sT, w1eT, w2T, b2)


def _node_mlp(nf, agg2, w1aT, w1bT, b1, w2T, b2, gamma, beta, interpret=False):
    def body(nf_ref, agg_ref, wa_ref, wb_ref, b1_ref, w2_ref, b2_ref, g_ref,
             be_ref, out_ref):
        x = nf_ref[...]
        agg = agg_ref[0] + agg_ref[1]
        u = jnp.maximum(
            jnp.dot(x, wa_ref[...], preferred_element_type=jnp.float32)
            + jnp.dot(agg, wb_ref[...], preferred_element_type=jnp.float32)
            + b1_ref[...],
            0.0,
        )
        u = jnp.dot(u, w2_ref[...], preferred_element_type=jnp.float32) + b2_ref[...]
        mean = jnp.mean(u, axis=0, keepdims=True)
        cen = u - mean
        var = jnp.mean(cen * cen, axis=0, keepdims=True)
        u = g_ref[...] * cen * lax.rsqrt(var + _EPS) + be_ref[...]
        out_ref[...] = x + u

    return pl.pallas_call(
        body,
        out_shape=jax.ShapeDtypeStruct((_N, _D), jnp.float32),
        interpret=interpret,
    )(nf, agg2, w1aT, w1bT, b1, w2T, b2, gamma, beta)


def kernel(node_features, edge_index, edge_features, eW1, eb1, eW2, eb2,
           nW1, nb1, nW2, nb2, gamma, beta):
    src = edge_index[0]
    dst = edge_index[1]
    idx_flat = jnp.concatenate([src, dst + _N])

    w1sT = eW1[:, :_D].T.astype(jnp.bfloat16)
    w1dT = eW1[:, _D:2 * _D].T
    w1eT = eW1[:, 2 * _D:].T.astype(jnp.bfloat16)
    table = _build_table(node_features, w1dT, eb1.reshape(1, _D))
    gathered = _sc_gather(table, idx_flat)

    msg = _edge_mlp(
        gathered, edge_features, w1sT, w1eT,
        eW2.T.astype(jnp.bfloat16), eb2.reshape(1, _D),
    )

    zeros_nd = jnp.zeros((_N, _D), jnp.float32)
    agg2 = _sc_scatter_add(msg, dst, zeros_nd)

    w1aT = nW1[:, :_D].T
    w1bT = nW1[:, _D:].T
    return _node_mlp(
        node_features, agg2, w1aT, w1bT,
        nb1.reshape(1, _D), nW2.T, nb2.reshape(1, _D),
        gamma.reshape(1, _D), beta.reshape(1, _D),
    )


# speedup vs baseline: 1.4809x; 1.0038x over previous
import functools

import jax
import jax.numpy as jnp
from jax import lax
from jax.experimental import pallas as pl
from jax.experimental.pallas import tpu as pltpu
from jax.experimental.pallas import tpu_sc as plsc

_N = 10000
_E = 320000
_D = 128
_ED = 16
_EPS = 1e-5

_NC = 2
_NS = 16
_GW = 256
_SW = 128
_EB = 8000
_NCHUNK = 2


def _sc_gather(table, idx_flat):
    n_idx = idx_flat.shape[0]
    width = table.shape[1]
    idx2 = idx_flat.reshape(1, n_idx)
    mesh = plsc.VectorSubcoreMesh(core_axis_name="c", subcore_axis_name="s")

    @functools.partial(
        pl.kernel,
        out_type=jax.ShapeDtypeStruct((n_idx, width), table.dtype),
        mesh=mesh,
        compiler_params=pltpu.CompilerParams(use_tc_tiling_on_sc=False),
    )
    def k(tab_hbm, i_hbm, o_hbm):
        def body(i_vmem, o_vmem):
            pltpu.sync_copy(tab_hbm.at[i_vmem.at[0]], o_vmem)

        pltpu.emit_pipeline(
            body,
            grid=(n_idx // _GW,),
            in_specs=[pl.BlockSpec((1, _GW), lambda i: (0, i))],
            out_specs=[pl.BlockSpec((_GW, width), lambda i: (i, 0))],
            core_axis_name=("c", "s"),
            dimension_semantics=(pltpu.PARALLEL,),
        )(i_hbm, o_hbm)

    return k(table, idx2)


def _pack_rows(x):
    xb = x.astype(jnp.bfloat16).astype(jnp.float32)
    bl = lax.bitcast_convert_type(xb[:, : _D // 2], jnp.int32)
    bh = lax.bitcast_convert_type(xb[:, _D // 2:], jnp.int32)
    return lax.shift_right_logical(bl, 16) | bh


def _unpack_halves(w):
    lo = lax.bitcast_convert_type(w << 16, jnp.float32)
    hi = lax.bitcast_convert_type(w & jnp.int32(-65536), jnp.float32)
    return lo, hi


def _sc_scatter_add(msg, dst, zeros_nd):
    ne = dst.shape[0]
    dst2 = dst.reshape(1, ne)
    mesh = plsc.VectorSubcoreMesh(core_axis_name="c", subcore_axis_name="s")
    stripe = 624

    @functools.partial(
        pl.kernel,
        out_type=jax.ShapeDtypeStruct((_NC, _N, _D), jnp.float32),
        mesh=mesh,
        scratch_types=[pltpu.VMEM_SHARED((_N, _D), jnp.float32)],
    )
    def k(msg_hbm, dst_hbm, z_hbm, o_hbm, acc):
        cid = lax.axis_index("c")
        sid = lax.axis_index("s")
        r0 = sid * stripe
        pltpu.sync_copy(z_hbm.at[pl.ds(r0, stripe)], acc.at[pl.ds(r0, stripe)])

        @pl.when(sid == _NS - 1)
        def _():
            t0 = _NS * stripe
            pltpu.sync_copy(
                z_hbm.at[pl.ds(t0, _N - _NS * stripe)],
                acc.at[pl.ds(t0, _N - _NS * stripe)],
            )

        plsc.subcore_barrier()

        def body(msg_vmem, dst_vmem):
            pltpu.sync_copy(msg_vmem, acc.at[dst_vmem.at[0]], add=True)

        pltpu.emit_pipeline(
            body,
            grid=(ne // _SW,),
            in_specs=[
                pl.BlockSpec((_SW, _D), lambda i: (i, 0)),
                pl.BlockSpec((1, _SW), lambda i: (0, i)),
            ],
            out_specs=[],
            core_axis_name=("c", "s"),
            dimension_semantics=(pltpu.PARALLEL,),
        )(msg_hbm, dst_hbm)
        plsc.subcore_barrier()
        pltpu.sync_copy(
            acc.at[pl.ds(r0, stripe)], o_hbm.at[cid].at[pl.ds(r0, stripe)]
        )

        @pl.when(sid == _NS - 1)
        def _():
            t0 = _NS * stripe
            pltpu.sync_copy(
                acc.at[pl.ds(t0, _N - _NS * stripe)],
                o_hbm.at[cid].at[pl.ds(t0, _N - _NS * stripe)],
            )

    return k(msg, dst2, zeros_nd)


def _build_table(nf, w1dT, b1, interpret=False):
    def body(nf_ref, w_ref, b_ref, out_ref):
        x = nf_ref[...]
        out_ref[pl.ds(0, _N), :] = _pack_rows(x)
        pd = (
            jnp.dot(x.astype(jnp.bfloat16), w_ref[...],
                    preferred_element_type=jnp.float32)
            + b_ref[...]
        )
        out_ref[pl.ds(_N, _N), :] = _pack_rows(pd)

    return pl.pallas_call(
        body,
        out_shape=jax.ShapeDtypeStruct((2 * _N, _D // 2), jnp.int32),
        interpret=interpret,
    )(nf, w1dT.astype(jnp.bfloat16), b1)


def _edge_mlp(gathered, ef, w1sT, w1eT, w2T, b2, interpret=False):
    ne = ef.shape[0]
    nb = ne // _EB

    hd = _D // 2

    def body(src_ref, pd_ref, ef_ref, w1s_ref, w1e_ref, w2_ref, b2_ref,
             out_ref):
        s_lo, s_hi = _unpack_halves(src_ref[...])
        p_lo, p_hi = _unpack_halves(pd_ref[...])
        slb = s_lo.astype(jnp.bfloat16)
        shb = s_hi.astype(jnp.bfloat16)
        efb = ef_ref[...].astype(jnp.bfloat16)

        def l1(c0):
            return (
                jnp.dot(slb, w1s_ref[pl.ds(0, hd), pl.ds(c0, hd)],
                        preferred_element_type=jnp.float32)
                + jnp.dot(shb, w1s_ref[pl.ds(hd, hd), pl.ds(c0, hd)],
                          preferred_element_type=jnp.float32)
                + jnp.dot(efb, w1e_ref[:, pl.ds(c0, hd)],
                          preferred_element_type=jnp.float32)
            )

        ha = jnp.maximum(l1(0) + p_lo, 0.0).astype(jnp.bfloat16)
        hb = jnp.maximum(l1(hd) + p_hi, 0.0).astype(jnp.bfloat16)

        def l2(c0):
            return (
                jnp.dot(ha, w2_ref[pl.ds(0, hd), pl.ds(c0, hd)],
                        preferred_element_type=jnp.float32)
                + jnp.dot(hb, w2_ref[pl.ds(hd, hd), pl.ds(c0, hd)],
                          preferred_element_type=jnp.float32)
                + b2_ref[:, pl.ds(c0, hd)]
            )

        out_ref[:, pl.ds(0, hd)] = s_lo * jax.nn.sigmoid(l2(0))
        out_ref[:, pl.ds(hd, hd)] = s_hi * jax.nn.sigmoid(l2(hd))

    return pl.pallas_call(
        body,
        grid=(nb,),
        in_specs=[
            pl.BlockSpec((_EB, _D // 2), lambda i: (i, 0)),
            pl.BlockSpec((_EB, _D // 2), lambda i: (i + nb, 0)),
            pl.BlockSpec((_EB, _ED), lambda i: (i, 0)),
            pl.BlockSpec((_D, _D), lambda i: (0, 0)),
            pl.BlockSpec((_ED, _D), lambda i: (0, 0)),
            pl.BlockSpec((_D, _D), lambda i: (0, 0)),
            pl.BlockSpec((1, _D), lambda i: (0, 0)),
        ],
        out_specs=pl.BlockSpec((_EB, _D), lambda i: (i, 0)),
        out_shape=jax.ShapeDtypeStruct((ne, _D), jnp.float32),
        interpret=interpret,
    )(gathered, gathered, ef, w1sT, w1eT, w2T, b2)


def _node_mlp(nf, parts, w1aT, w1bT, b1, w2T, b2, gamma, beta, interpret=False):
    nparts = len(parts)

    def body(nf_ref, *refs):
        (agg_refs, (wa_ref, wb_ref, b1_ref, w2_ref, b2_ref, g_ref, be_ref,
                    out_ref)) = refs[:nparts], refs[nparts:]
        x = nf_ref[...]
        agg = agg_refs[0][0] + agg_refs[0][1]
        for p in range(1, nparts):
            agg = agg + agg_refs[p][0] + agg_refs[p][1]
        u = jnp.maximum(
            jnp.dot(x, wa_ref[...], preferred_element_type=jnp.float32)
            + jnp.dot(agg, wb_ref[...], preferred_element_type=jnp.float32)
            + b1_ref[...],
            0.0,
        )
        u = jnp.dot(u, w2_ref[...], preferred_element_type=jnp.float32) + b2_ref[...]
        mean = jnp.mean(u, axis=0, keepdims=True)
        cen = u - mean
        var = jnp.mean(cen * cen, axis=0, keepdims=True)
        u = g_ref[...] * cen * lax.rsqrt(var + _EPS) + be_ref[...]
        out_ref[...] = x + u

    return pl.pallas_call(
        body,
        out_shape=jax.ShapeDtypeStruct((_N, _D), jnp.float32),
        interpret=interpret,
    )(nf, *parts, w1aT, w1bT, b1, w2T, b2, gamma, beta)


def kernel(node_features, edge_index, edge_features, eW1, eb1, eW2, eb2,
           nW1, nb1, nW2, nb2, gamma, beta):
    src = edge_index[0]
    dst = edge_index[1]

    w1sT = eW1[:, :_D].T.astype(jnp.bfloat16)
    w1dT = eW1[:, _D:2 * _D].T
    w1eT = eW1[:, 2 * _D:].T.astype(jnp.bfloat16)
    w2eT = eW2.T.astype(jnp.bfloat16)
    eb2r = eb2.reshape(1, _D)
    table = _build_table(node_features, w1dT, eb1.reshape(1, _D))
    zeros_nd = jnp.zeros((_N, _D), jnp.float32)

    ec = _E // _NCHUNK
    partials = []
    for k in range(_NCHUNK):
        sl = slice(k * ec, (k + 1) * ec)
        idx_k = jnp.concatenate([src[sl], dst[sl] + _N])
        gathered = _sc_gather(table, idx_k)
        msg = _edge_mlp(gathered, edge_features[sl], w1sT, w1eT, w2eT, eb2r)
        partials.append(_sc_scatter_add(msg, dst[sl], zeros_nd))

    w1aT = nW1[:, :_D].T
    w1bT = nW1[:, _D:].T
    return _node_mlp(
        node_features, partials, w1aT, w1bT,
        nb1.reshape(1, _D), nW2.T, nb2.reshape(1, _D),
        gamma.reshape(1, _D), beta.reshape(1, _D),
    )
